# Initial kernel scaffold; baseline (speedup 1.0000x reference)
import jax
import jax.numpy as jnp
from jax.experimental import pallas as pl


def kernel(x, edge_index, edge_attr, W1, b1, Wmu, bmu, Wlv, blv):
    raise NotImplementedError("write your pallas kernel here")


# speedup vs baseline: 11.2464x; 11.2464x over previous
import functools

import jax
import jax.numpy as jnp
from jax import lax
from jax.experimental import pallas as pl
from jax.experimental.pallas import tpu as pltpu
from jax.experimental.pallas import tpu_sc as plsc

N = 10000
E = 320000
D = 128
L = 16
NC = 2
NS = 16
NW = NC * NS
KE = 128
NCHUNK = E // KE
FULL_T = NCHUNK // NW
REM = NCHUNK - FULL_T * NW
NPAD = 10240
RPT = NPAD // NS

_mesh = plsc.VectorSubcoreMesh(core_axis_name="c", subcore_axis_name="s")
_params = pltpu.CompilerParams(needs_layout_passes=False)


def _splat(ewv, j):
    return plsc.load_gather(ewv, [jnp.full((L,), j, jnp.int32)])


GPT = NPAD // 8 // NS


@functools.partial(
    pl.kernel,
    out_type=jax.ShapeDtypeStruct((NC, NPAD // 8, D), jnp.float32),
    mesh=_mesh,
    compiler_params=_params,
    scratch_types=[
        pltpu.VMEM((KE,), jnp.int32),
        pltpu.VMEM((KE,), jnp.int32),
        pltpu.VMEM((KE,), jnp.float32),
        pltpu.VMEM((KE, D), jnp.float32),
        pltpu.VMEM_SHARED((NPAD // 8, D), jnp.float32),
    ],
)
def _sc_degree(dst_hbm, ew_hbm, out_hbm, dstv, dst8v, ewv, rows, acc):
    c = lax.axis_index("c")
    s = lax.axis_index("s")
    w = c * NS + s

    @pl.loop(0, KE)
    def _(i):
        for r in range(D // L):
            rows[i, pl.ds(r * L, L)] = jnp.zeros((L,), jnp.float32)

    pltpu.sync_copy(rows.at[pl.ds(0, GPT)], acc.at[pl.ds(s * GPT, GPT)])
    plsc.subcore_barrier()

    def do_chunk(t):
        base = (w + t * NW) * KE
        pltpu.sync_copy(dst_hbm.at[pl.ds(base, KE)], dstv)
        pltpu.sync_copy(ew_hbm.at[pl.ds(base, KE)], ewv)

        @pl.loop(0, KE // L)
        def _(g):
            gofs = pl.ds(g * L, L)
            dst8v[gofs] = lax.shift_right_logical(dstv[gofs], 3)

        @pl.loop(0, KE)
        def _(j):
            spl = _splat(ewv, j)
            dmod = jnp.bitwise_and(plsc.load_gather(
                dstv, [jnp.full((L,), j, jnp.int32)]), 7)
            for r in range(D // L):
                rows[j, pl.ds(r * L, L)] = jnp.where(
                    dmod == r, spl, jnp.zeros((L,), jnp.float32))

        pltpu.sync_copy(rows, acc.at[dst8v], add=True)

    @pl.loop(0, FULL_T)
    def _(t):
        do_chunk(t)

    @pl.when(w < REM)
    def _():
        do_chunk(FULL_T)

    plsc.subcore_barrier()
    pltpu.sync_copy(acc.at[pl.ds(s * GPT, GPT)],
                    out_hbm.at[c, pl.ds(s * GPT, GPT)])


@functools.partial(
    pl.kernel,
    out_type=jax.ShapeDtypeStruct((NC, NPAD, D), jnp.float32),
    mesh=_mesh,
    compiler_params=_params,
    scratch_types=[
        pltpu.VMEM((KE,), jnp.int32),
        pltpu.VMEM((KE,), jnp.int32),
        pltpu.VMEM((KE,), jnp.float32),
        pltpu.VMEM((KE, D), jnp.float32),
        pltpu.VMEM_SHARED((NPAD, D), jnp.float32),
        pltpu.SemaphoreType.DMA,
    ],
)
def _sc_propagate(y_hbm, src_hbm, dst_hbm, ew_hbm, out_hbm,
                  srcv, dstv, ewv, rows, acc, sem):
    c = lax.axis_index("c")
    s = lax.axis_index("s")
    w = c * NS + s

    @pl.loop(0, KE)
    def _(i):
        for r in range(D // L):
            rows[i, pl.ds(r * L, L)] = jnp.zeros((L,), jnp.float32)

    @pl.loop(0, RPT // KE)
    def _(k):
        pltpu.sync_copy(rows, acc.at[pl.ds(s * RPT + k * KE, KE)])

    plsc.subcore_barrier()

    def do_chunk(t):
        base = (w + t * NW) * KE
        pltpu.sync_copy(src_hbm.at[pl.ds(base, KE)], srcv)
        pltpu.sync_copy(dst_hbm.at[pl.ds(base, KE)], dstv)
        pltpu.sync_copy(ew_hbm.at[pl.ds(base, KE)], ewv)
        pltpu.async_copy(y_hbm.at[srcv], rows, sem).wait()

        @pl.loop(0, KE)
        def _(j):
            spl = _splat(ewv, j)
            for r in range(D // L):
                rows[j, pl.ds(r * L, L)] = rows[j, pl.ds(r * L, L)] * spl

        pltpu.sync_copy(rows, acc.at[dstv], add=True)

    @pl.loop(0, FULL_T)
    def _(t):
        do_chunk(t)

    @pl.when(w < REM)
    def _():
        do_chunk(FULL_T)

    plsc.subcore_barrier()

    @pl.loop(0, RPT // KE)
    def _(k):
        off = s * RPT + k * KE
        pltpu.sync_copy(acc.at[pl.ds(off, KE)], out_hbm.at[c, pl.ds(off, KE)])


def _tc_stage1_body(degp_ref, x_ref, w1_ref, y_ref, dis_ref):
    deg = degp_ref[0] + degp_ref[1] + 1.0
    dis = jnp.where(deg > 0, lax.rsqrt(jnp.maximum(deg, 1e-12)), 0.0)
    y = jnp.dot(x_ref[...], w1_ref[...], preferred_element_type=jnp.float32)
    y_ref[...] = y * dis
    dis_ref[...] = dis


def _tc_stage2_body(agg_ref, y_ref, dis_ref, b1_ref, w2_ref, y2_ref):
    dis = dis_ref[...]
    h = dis * (agg_ref[0, :N] + agg_ref[1, :N] + y_ref[...]) + b1_ref[...]
    h = jnp.maximum(h, 0.0)
    y2 = jnp.dot(h, w2_ref[...], preferred_element_type=jnp.float32)
    y2_ref[...] = y2 * dis


def _tc_stage3_body(agg_ref, y2_ref, dis_ref, bcat_ref, out_ref):
    dis = dis_ref[...]
    out_ref[...] = dis * (agg_ref[0, :N] + agg_ref[1, :N] + y2_ref[...]) \
        + bcat_ref[...]


def _tc_stage1(deg_parts, x, W1):
    return pl.pallas_call(
        _tc_stage1_body,
        out_shape=(
            jax.ShapeDtypeStruct((N, D), jnp.float32),
            jax.ShapeDtypeStruct((N, 1), jnp.float32),
        ),
    )(deg_parts, x, W1)


def _tc_stage2(agg, y, dis, b1, W2):
    return pl.pallas_call(
        _tc_stage2_body,
        out_shape=jax.ShapeDtypeStruct((N, D), jnp.float32),
    )(agg, y, dis, b1, W2)


def _tc_stage3(agg, y2, dis, bcat):
    return pl.pallas_call(
        _tc_stage3_body,
        out_shape=jax.ShapeDtypeStruct((N, D), jnp.float32),
    )(agg, y2, dis, bcat)


@jax.jit
def kernel(x, edge_index, edge_attr, W1, b1, Wmu, bmu, Wlv, blv):
    src = edge_index[0]
    dst = edge_index[1]
    W2 = jnp.concatenate([Wmu, Wlv], axis=1)
    bcat = jnp.concatenate([bmu, blv])[None, :]

    deg_parts = _sc_degree(dst, edge_attr)
    deg_lin = deg_parts[:, :, ::L].reshape(NC, NPAD)[:, :N, None]
    y1, dis = _tc_stage1(deg_lin, x, W1)
    agg1 = _sc_propagate(y1, src, dst, edge_attr)
    y2 = _tc_stage2(agg1, y1, dis, b1[None, :], W2)
    agg2 = _sc_propagate(y2, src, dst, edge_attr)
    out2 = _tc_stage3(agg2, y2, dis, bcat)
    return out2[:, :D // 2], out2[:, D // 2:]


# speedup vs baseline: 16.4729x; 1.4647x over previous
import functools

import jax
import jax.numpy as jnp
from jax import lax
from jax.experimental import pallas as pl
from jax.experimental.pallas import tpu as pltpu
from jax.experimental.pallas import tpu_sc as plsc

N = 10000
E = 320000
D = 128
L = 16
NC = 2
NS = 16
NW = NC * NS
KE = 128
NCHUNK = E // KE
FULL_T = NCHUNK // NW
REM = NCHUNK - FULL_T * NW
NPAD = 10240
RPT = NPAD // NS

_mesh = plsc.VectorSubcoreMesh(core_axis_name="c", subcore_axis_name="s")
_params = pltpu.CompilerParams(needs_layout_passes=False)


def _splat(ewv, j):
    return plsc.load_gather(ewv, [jnp.full((L,), j, jnp.int32)])


GPT = NPAD // 8 // NS


@functools.partial(
    pl.kernel,
    out_type=jax.ShapeDtypeStruct((NC, NPAD // 8, D), jnp.float32),
    mesh=_mesh,
    compiler_params=_params,
    scratch_types=[
        pltpu.VMEM((KE,), jnp.int32),
        pltpu.VMEM((KE,), jnp.int32),
        pltpu.VMEM((KE,), jnp.float32),
        pltpu.VMEM((KE, D), jnp.float32),
        pltpu.VMEM_SHARED((NPAD // 8, D), jnp.float32),
    ],
)
def _sc_degree(dst_hbm, ew_hbm, out_hbm, dstv, dst8v, ewv, rows, acc):
    c = lax.axis_index("c")
    s = lax.axis_index("s")
    w = c * NS + s

    @pl.loop(0, KE)
    def _(i):
        for r in range(D // L):
            rows[i, pl.ds(r * L, L)] = jnp.zeros((L,), jnp.float32)

    pltpu.sync_copy(rows.at[pl.ds(0, GPT)], acc.at[pl.ds(s * GPT, GPT)])
    plsc.subcore_barrier()

    def do_chunk(t):
        base = (w + t * NW) * KE
        pltpu.sync_copy(dst_hbm.at[pl.ds(base, KE)], dstv)
        pltpu.sync_copy(ew_hbm.at[pl.ds(base, KE)], ewv)

        @pl.loop(0, KE // L)
        def _(g):
            gofs = pl.ds(g * L, L)
            dst8v[gofs] = lax.shift_right_logical(dstv[gofs], 3)

        @pl.loop(0, KE)
        def _(j):
            spl = _splat(ewv, j)
            dmod = jnp.bitwise_and(plsc.load_gather(
                dstv, [jnp.full((L,), j, jnp.int32)]), 7)
            for r in range(D // L):
                rows[j, pl.ds(r * L, L)] = jnp.where(
                    dmod == r, spl, jnp.zeros((L,), jnp.float32))

        pltpu.sync_copy(rows, acc.at[dst8v], add=True)

    @pl.loop(0, FULL_T)
    def _(t):
        do_chunk(t)

    @pl.when(w < REM)
    def _():
        do_chunk(FULL_T)

    plsc.subcore_barrier()
    pltpu.sync_copy(acc.at[pl.ds(s * GPT, GPT)],
                    out_hbm.at[c, pl.ds(s * GPT, GPT)])


EPW = E // NW
NFULL = EPW // KE
TAIL = EPW - NFULL * KE


@functools.partial(
    pl.kernel,
    out_type=jax.ShapeDtypeStruct((NC, NPAD, D), jnp.float32),
    mesh=_mesh,
    compiler_params=_params,
    scratch_types=[
        pltpu.VMEM((KE,), jnp.int32),
        pltpu.VMEM((KE,), jnp.int32),
        pltpu.VMEM((KE,), jnp.int32),
        pltpu.VMEM((KE,), jnp.int32),
        pltpu.VMEM((KE,), jnp.float32),
        pltpu.VMEM((KE,), jnp.float32),
        pltpu.VMEM((TAIL,), jnp.int32),
        pltpu.VMEM((TAIL,), jnp.int32),
        pltpu.VMEM((TAIL,), jnp.float32),
        pltpu.VMEM((KE, D), jnp.float32),
        pltpu.VMEM((KE, D), jnp.float32),
        pltpu.SemaphoreType.DMA,
        pltpu.SemaphoreType.DMA,
        pltpu.SemaphoreType.DMA,
        pltpu.SemaphoreType.DMA,
        pltpu.SemaphoreType.DMA,
        pltpu.VMEM_SHARED((NPAD, D), jnp.float32),
    ],
)
def _sc_propagate(y_hbm, src_hbm, dst_hbm, ew_hbm, out_hbm,
                  srcA, srcB, dstA, dstB, ewA, ewB,
                  srcT, dstT, ewT, rowsA, rowsB,
                  isemA, isemB, gsemA, gsemB, semT, acc):
    c = lax.axis_index("c")
    s = lax.axis_index("s")
    w = c * NS + s
    e0 = w * EPW

    @pl.loop(0, KE)
    def _(i):
        for r in range(D // L):
            rowsA[i, pl.ds(r * L, L)] = jnp.zeros((L,), jnp.float32)

    @pl.loop(0, RPT // KE)
    def _(k):
        pltpu.sync_copy(rowsA, acc.at[pl.ds(s * RPT + k * KE, KE)])

    plsc.subcore_barrier()

    bufs = ((srcA, dstA, ewA, rowsA, isemA, gsemA),
            (srcB, dstB, ewB, rowsB, isemB, gsemB))

    def start_chunk(b, t):
        srcc, dstc, ewc, rows, isem, gsem = bufs[b]
        base = e0 + t * KE
        csrc = pltpu.make_async_copy(src_hbm.at[pl.ds(base, KE)], srcc, isem)
        cdst = pltpu.make_async_copy(dst_hbm.at[pl.ds(base, KE)], dstc, isem)
        cew = pltpu.make_async_copy(ew_hbm.at[pl.ds(base, KE)], ewc, isem)
        csrc.start()
        cdst.start()
        cew.start()
        csrc.wait()
        cdst.wait()
        cew.wait()
        pltpu.async_copy(y_hbm.at[srcc], rows, gsem)

    def finish_chunk(b, t):
        srcc, dstc, ewc, rows, isem, gsem = bufs[b]
        pltpu.make_async_copy(y_hbm.at[srcc], rows, gsem).wait()

        @pl.loop(0, KE)
        def _(j):
            spl = _splat(ewc, j)
            for r in range(D // L):
                rows[j, pl.ds(r * L, L)] = rows[j, pl.ds(r * L, L)] * spl

        pltpu.sync_copy(rows, acc.at[dstc], add=True)

    start_chunk(0, 0)
    start_chunk(1, 1)

    @pl.loop(0, NFULL, step=2)
    def _(t):
        for b in (0, 1):
            tt = t + b
            finish_chunk(b, tt)

            @pl.when(tt + 2 < NFULL)
            def _():
                start_chunk(b, tt + 2)

    tbase = e0 + NFULL * KE
    pltpu.sync_copy(src_hbm.at[pl.ds(tbase, TAIL)], srcT)
    pltpu.sync_copy(dst_hbm.at[pl.ds(tbase, TAIL)], dstT)
    pltpu.sync_copy(ew_hbm.at[pl.ds(tbase, TAIL)], ewT)
    rowsT = rowsA.at[pl.ds(0, TAIL)]
    pltpu.async_copy(y_hbm.at[srcT], rowsT, semT).wait()

    @pl.loop(0, TAIL)
    def _(j):
        spl = _splat(ewT, j)
        for r in range(D // L):
            rowsA[j, pl.ds(r * L, L)] = rowsA[j, pl.ds(r * L, L)] * spl

    pltpu.sync_copy(rowsT, acc.at[dstT], add=True)

    plsc.subcore_barrier()

    @pl.loop(0, RPT // KE)
    def _(k):
        off = s * RPT + k * KE
        pltpu.sync_copy(acc.at[pl.ds(off, KE)], out_hbm.at[c, pl.ds(off, KE)])


def _tc_stage1_body(degp_ref, x_ref, w1_ref, y_ref, dis_ref):
    deg = degp_ref[0] + degp_ref[1] + 1.0
    dis = jnp.where(deg > 0, lax.rsqrt(jnp.maximum(deg, 1e-12)), 0.0)
    y = jnp.dot(x_ref[...], w1_ref[...], preferred_element_type=jnp.float32)
    y_ref[...] = y * dis
    dis_ref[...] = dis


def _tc_stage2_body(agg_ref, y_ref, dis_ref, b1_ref, w2_ref, y2_ref):
    dis = dis_ref[...]
    h = dis * (agg_ref[0, :N] + agg_ref[1, :N] + y_ref[...]) + b1_ref[...]
    h = jnp.maximum(h, 0.0)
    y2 = jnp.dot(h, w2_ref[...], preferred_element_type=jnp.float32)
    y2_ref[...] = y2 * dis


def _tc_stage3_body(agg_ref, y2_ref, dis_ref, bcat_ref, out_ref):
    dis = dis_ref[...]
    out_ref[...] = dis * (agg_ref[0, :N] + agg_ref[1, :N] + y2_ref[...]) \
        + bcat_ref[...]


def _tc_stage1(deg_parts, x, W1):
    return pl.pallas_call(
        _tc_stage1_body,
        out_shape=(
            jax.ShapeDtypeStruct((N, D), jnp.float32),
            jax.ShapeDtypeStruct((N, 1), jnp.float32),
        ),
    )(deg_parts, x, W1)


def _tc_stage2(agg, y, dis, b1, W2):
    return pl.pallas_call(
        _tc_stage2_body,
        out_shape=jax.ShapeDtypeStruct((N, D), jnp.float32),
    )(agg, y, dis, b1, W2)


def _tc_stage3(agg, y2, dis, bcat):
    return pl.pallas_call(
        _tc_stage3_body,
        out_shape=jax.ShapeDtypeStruct((N, D), jnp.float32),
    )(agg, y2, dis, bcat)


@jax.jit
def kernel(x, edge_index, edge_attr, W1, b1, Wmu, bmu, Wlv, blv):
    src = edge_index[0]
    dst = edge_index[1]
    W2 = jnp.concatenate([Wmu, Wlv], axis=1)
    bcat = jnp.concatenate([bmu, blv])[None, :]

    deg_parts = _sc_degree(dst, edge_attr)
    deg_lin = deg_parts[:, :, ::L].reshape(NC, NPAD)[:, :N, None]
    y1, dis = _tc_stage1(deg_lin, x, W1)
    agg1 = _sc_propagate(y1, src, dst, edge_attr)
    y2 = _tc_stage2(agg1, y1, dis, b1[None, :], W2)
    agg2 = _sc_propagate(y2, src, dst, edge_attr)
    out2 = _tc_stage3(agg2, y2, dis, bcat)
    return out2[:, :D // 2], out2[:, D // 2:]


# speedup vs baseline: 22.3462x; 1.3565x over previous
import functools

import jax
import jax.numpy as jnp
from jax import lax
from jax.experimental import pallas as pl
from jax.experimental.pallas import tpu as pltpu
from jax.experimental.pallas import tpu_sc as plsc

N = 10000
E = 320000
D = 128
L = 16
NC = 2
NS = 16
NW = NC * NS
KE = 128
NCHUNK = E // KE
FULL_T = NCHUNK // NW
REM = NCHUNK - FULL_T * NW
NPAD = 10240
RPT = NPAD // NS
EPW = E // NW
NFULL = EPW // KE
TAIL = EPW - NFULL * KE

_mesh = plsc.VectorSubcoreMesh(core_axis_name="c", subcore_axis_name="s")
_params = pltpu.CompilerParams(needs_layout_passes=False)


def _splat(ewv, j):
    return plsc.load_gather(ewv, [jnp.full((L,), j, jnp.int32)])


NROW = NPAD // D


@functools.partial(
    pl.kernel,
    out_type=jax.ShapeDtypeStruct((NC, NROW, D), jnp.float32),
    mesh=_mesh,
    compiler_params=_params,
    scratch_types=[
        pltpu.VMEM((EPW,), jnp.int32),
        pltpu.VMEM((EPW,), jnp.float32),
        pltpu.VMEM((N * 8,), jnp.float32),
        pltpu.VMEM((NROW, D), jnp.float32),
        pltpu.VMEM((NROW,), jnp.int32),
        pltpu.VMEM_SHARED((NROW, D), jnp.float32),
    ],
)
def _sc_degree(dst_hbm, ew_hbm, out_hbm, dstall, ewall, acc8, stag, idv, acc):
    c = lax.axis_index("c")
    s = lax.axis_index("s")
    w = c * NS + s
    e0 = w * EPW

    pltpu.sync_copy(dst_hbm.at[pl.ds(e0, EPW)], dstall)
    pltpu.sync_copy(ew_hbm.at[pl.ds(e0, EPW)], ewall)

    zero16 = jnp.zeros((L,), jnp.float32)
    lane = lax.broadcasted_iota(jnp.int32, (L,), 0)

    @pl.loop(0, N * 8 // L)
    def _(i):
        acc8[pl.ds(i * L, L)] = zero16

    @pl.loop(0, NROW)
    def _(q):
        for r in range(D // L):
            stag[q, pl.ds(r * L, L)] = zero16

    @pl.loop(0, NROW // L)
    def _(g):
        idv[pl.ds(g * L, L)] = lane + g * L

    @pl.when(s == 0)
    def _():
        pltpu.sync_copy(stag, acc)

    plsc.subcore_barrier()

    m_lo = lane < 8
    m_hi = lane >= 8
    col_lo = lane
    col_hi = lane - 8

    @pl.loop(0, EPW // L)
    def _(g):
        gofs = pl.ds(g * L, L)
        dstv = dstall[gofs]
        ewv = ewall[gofs]
        base8 = dstv * 8
        plsc.addupdate_scatter(acc8, [base8 + col_lo], ewv, mask=m_lo)
        plsc.addupdate_scatter(acc8, [base8 + col_hi], ewv, mask=m_hi)

    @pl.loop(0, N // L)
    def _(m):
        tot = zero16
        for cc in range(8):
            tot = tot + plsc.load_gather(acc8, [(lane + m * L) * 8 + cc])
        stag[m // 8, pl.ds((m % 8) * L, L)] = tot

    pltpu.sync_copy(stag, acc.at[idv], add=True)
    plsc.subcore_barrier()

    @pl.when(s < NROW // 8)
    def _():
        pltpu.sync_copy(acc.at[pl.ds(s * 8, 8)], out_hbm.at[c, pl.ds(s * 8, 8)])


@functools.partial(
    pl.kernel,
    out_type=jax.ShapeDtypeStruct((NC, NPAD, D), jnp.float32),
    mesh=_mesh,
    compiler_params=_params,
    scratch_types=[
        pltpu.VMEM((KE,), jnp.int32),
        pltpu.VMEM((KE,), jnp.int32),
        pltpu.VMEM((KE,), jnp.int32),
        pltpu.VMEM((KE,), jnp.int32),
        pltpu.VMEM((KE,), jnp.float32),
        pltpu.VMEM((KE,), jnp.float32),
        pltpu.VMEM((TAIL,), jnp.int32),
        pltpu.VMEM((TAIL,), jnp.int32),
        pltpu.VMEM((TAIL,), jnp.float32),
        pltpu.VMEM((KE, D), jnp.float32),
        pltpu.VMEM((KE, D), jnp.float32),
        pltpu.SemaphoreType.DMA,
        pltpu.SemaphoreType.DMA,
        pltpu.SemaphoreType.DMA,
        pltpu.SemaphoreType.DMA,
        pltpu.SemaphoreType.DMA,
        pltpu.VMEM_SHARED((NPAD, D), jnp.float32),
    ],
)
def _sc_propagate(y_hbm, src_hbm, dst_hbm, ew_hbm, out_hbm,
                  srcA, srcB, dstA, dstB, ewA, ewB,
                  srcT, dstT, ewT, rowsA, rowsB,
                  isemA, isemB, gsemA, gsemB, semT, acc):
    c = lax.axis_index("c")
    s = lax.axis_index("s")
    w = c * NS + s
    e0 = w * EPW

    @pl.loop(0, KE)
    def _(i):
        for r in range(D // L):
            rowsA[i, pl.ds(r * L, L)] = jnp.zeros((L,), jnp.float32)

    @pl.loop(0, RPT // KE)
    def _(k):
        pltpu.sync_copy(rowsA, acc.at[pl.ds(s * RPT + k * KE, KE)])

    plsc.subcore_barrier()

    bufs = ((srcA, dstA, ewA, rowsA, isemA, gsemA),
            (srcB, dstB, ewB, rowsB, isemB, gsemB))

    def start_chunk(b, t):
        srcc, dstc, ewc, rows, isem, gsem = bufs[b]
        base = e0 + t * KE
        csrc = pltpu.make_async_copy(src_hbm.at[pl.ds(base, KE)], srcc, isem)
        cdst = pltpu.make_async_copy(dst_hbm.at[pl.ds(base, KE)], dstc, isem)
        cew = pltpu.make_async_copy(ew_hbm.at[pl.ds(base, KE)], ewc, isem)
        csrc.start()
        cdst.start()
        cew.start()
        csrc.wait()
        cdst.wait()
        cew.wait()
        pltpu.async_copy(y_hbm.at[srcc], rows, gsem)

    def finish_chunk(b, t):
        srcc, dstc, ewc, rows, isem, gsem = bufs[b]
        pltpu.make_async_copy(y_hbm.at[srcc], rows, gsem).wait()

        @pl.loop(0, KE)
        def _(j):
            spl = _splat(ewc, j)
            for r in range(D // L):
                rows[j, pl.ds(r * L, L)] = rows[j, pl.ds(r * L, L)] * spl

        pltpu.sync_copy(rows, acc.at[dstc], add=True)

    start_chunk(0, 0)
    start_chunk(1, 1)

    @pl.loop(0, NFULL, step=2)
    def _(t):
        for b in (0, 1):
            tt = t + b
            finish_chunk(b, tt)

            @pl.when(tt + 2 < NFULL)
            def _():
                start_chunk(b, tt + 2)

    tbase = e0 + NFULL * KE
    pltpu.sync_copy(src_hbm.at[pl.ds(tbase, TAIL)], srcT)
    pltpu.sync_copy(dst_hbm.at[pl.ds(tbase, TAIL)], dstT)
    pltpu.sync_copy(ew_hbm.at[pl.ds(tbase, TAIL)], ewT)
    rowsT = rowsA.at[pl.ds(0, TAIL)]
    pltpu.async_copy(y_hbm.at[srcT], rowsT, semT).wait()

    @pl.loop(0, TAIL)
    def _(j):
        spl = _splat(ewT, j)
        for r in range(D // L):
            rowsA[j, pl.ds(r * L, L)] = rowsA[j, pl.ds(r * L, L)] * spl

    pltpu.sync_copy(rowsT, acc.at[dstT], add=True)

    plsc.subcore_barrier()

    @pl.loop(0, RPT // KE)
    def _(k):
        off = s * RPT + k * KE
        pltpu.sync_copy(acc.at[pl.ds(off, KE)], out_hbm.at[c, pl.ds(off, KE)])


def _tc_stage1_body(degp_ref, x_ref, w1_ref, y_ref, dis_ref):
    deg = degp_ref[0] + degp_ref[1] + 1.0
    dis = jnp.where(deg > 0, lax.rsqrt(jnp.maximum(deg, 1e-12)), 0.0)
    y = jnp.dot(x_ref[...], w1_ref[...], preferred_element_type=jnp.float32)
    y_ref[...] = y * dis
    dis_ref[...] = dis


def _tc_stage2_body(agg_ref, y_ref, dis_ref, b1_ref, w2_ref, y2_ref):
    dis = dis_ref[...]
    h = dis * (agg_ref[0, :N] + agg_ref[1, :N] + y_ref[...]) + b1_ref[...]
    h = jnp.maximum(h, 0.0)
    y2 = jnp.dot(h, w2_ref[...], preferred_element_type=jnp.float32)
    y2_ref[...] = y2 * dis


def _tc_stage3_body(agg_ref, y2_ref, dis_ref, bcat_ref, out_ref):
    dis = dis_ref[...]
    out_ref[...] = dis * (agg_ref[0, :N] + agg_ref[1, :N] + y2_ref[...]) \
        + bcat_ref[...]


def _tc_stage1(deg_parts, x, W1):
    return pl.pallas_call(
        _tc_stage1_body,
        out_shape=(
            jax.ShapeDtypeStruct((N, D), jnp.float32),
            jax.ShapeDtypeStruct((N, 1), jnp.float32),
        ),
    )(deg_parts, x, W1)


def _tc_stage2(agg, y, dis, b1, W2):
    return pl.pallas_call(
        _tc_stage2_body,
        out_shape=jax.ShapeDtypeStruct((N, D), jnp.float32),
    )(agg, y, dis, b1, W2)


def _tc_stage3(agg, y2, dis, bcat):
    return pl.pallas_call(
        _tc_stage3_body,
        out_shape=jax.ShapeDtypeStruct((N, D), jnp.float32),
    )(agg, y2, dis, bcat)


@jax.jit
def kernel(x, edge_index, edge_attr, W1, b1, Wmu, bmu, Wlv, blv):
    src = edge_index[0]
    dst = edge_index[1]
    W2 = jnp.concatenate([Wmu, Wlv], axis=1)
    bcat = jnp.concatenate([bmu, blv])[None, :]

    deg_parts = _sc_degree(dst, edge_attr)
    deg_lin = deg_parts.reshape(NC, NPAD)[:, :N, None]
    y1, dis = _tc_stage1(deg_lin, x, W1)
    agg1 = _sc_propagate(y1, src, dst, edge_attr)
    y2 = _tc_stage2(agg1, y1, dis, b1[None, :], W2)
    agg2 = _sc_propagate(y2, src, dst, edge_attr)
    out2 = _tc_stage3(agg2, y2, dis, bcat)
    return out2[:, :D // 2], out2[:, D // 2:]


# speedup vs baseline: 22.3991x; 1.0024x over previous
import functools

import jax
import jax.numpy as jnp
from jax import lax
from jax.experimental import pallas as pl
from jax.experimental.pallas import tpu as pltpu
from jax.experimental.pallas import tpu_sc as plsc

N = 10000
E = 320000
D = 128
L = 16
NC = 2
NS = 16
NW = NC * NS
KE = 128
NCHUNK = E // KE
FULL_T = NCHUNK // NW
REM = NCHUNK - FULL_T * NW
NPAD = 10240
RPT = NPAD // NS
EPW = E // NW
NFULL = EPW // KE
TAIL = EPW - NFULL * KE

_mesh = plsc.VectorSubcoreMesh(core_axis_name="c", subcore_axis_name="s")
_params = pltpu.CompilerParams(needs_layout_passes=False)


def _splat(ewv, j):
    return plsc.load_gather(ewv, [jnp.full((L,), j, jnp.int32)])


NROW = NPAD // D


@functools.partial(
    pl.kernel,
    out_type=jax.ShapeDtypeStruct((NC, NROW, D), jnp.float32),
    mesh=_mesh,
    compiler_params=_params,
    scratch_types=[
        pltpu.VMEM((EPW,), jnp.int32),
        pltpu.VMEM((EPW,), jnp.float32),
        pltpu.VMEM((N * 8,), jnp.float32),
        pltpu.VMEM((NROW, D), jnp.float32),
        pltpu.VMEM((NROW,), jnp.int32),
        pltpu.VMEM_SHARED((NROW, D), jnp.float32),
    ],
)
def _sc_degree(dst_hbm, ew_hbm, out_hbm, dstall, ewall, acc8, stag, idv, acc):
    c = lax.axis_index("c")
    s = lax.axis_index("s")
    w = c * NS + s
    e0 = w * EPW

    pltpu.sync_copy(dst_hbm.at[pl.ds(e0, EPW)], dstall)
    pltpu.sync_copy(ew_hbm.at[pl.ds(e0, EPW)], ewall)

    zero16 = jnp.zeros((L,), jnp.float32)
    lane = lax.broadcasted_iota(jnp.int32, (L,), 0)

    @pl.loop(0, N * 8 // L)
    def _(i):
        acc8[pl.ds(i * L, L)] = zero16

    @pl.loop(0, NROW)
    def _(q):
        for r in range(D // L):
            stag[q, pl.ds(r * L, L)] = zero16

    @pl.loop(0, NROW // L)
    def _(g):
        idv[pl.ds(g * L, L)] = lane + g * L

    @pl.when(s == 0)
    def _():
        pltpu.sync_copy(stag, acc)

    plsc.subcore_barrier()

    m_lo = lane < 8
    m_hi = lane >= 8
    col_lo = lane
    col_hi = lane - 8

    @pl.loop(0, EPW // L)
    def _(g):
        gofs = pl.ds(g * L, L)
        dstv = dstall[gofs]
        ewv = ewall[gofs]
        base8 = dstv * 8
        plsc.addupdate_scatter(acc8, [base8 + col_lo], ewv, mask=m_lo)
        plsc.addupdate_scatter(acc8, [base8 + col_hi], ewv, mask=m_hi)

    @pl.loop(0, N // L)
    def _(m):
        tot = zero16
        for cc in range(8):
            tot = tot + plsc.load_gather(acc8, [(lane + m * L) * 8 + cc])
        stag[m // 8, pl.ds((m % 8) * L, L)] = tot

    pltpu.sync_copy(stag, acc.at[idv], add=True)
    plsc.subcore_barrier()

    @pl.when(s < NROW // 8)
    def _():
        pltpu.sync_copy(acc.at[pl.ds(s * 8, 8)], out_hbm.at[c, pl.ds(s * 8, 8)])


@functools.partial(
    pl.kernel,
    out_type=jax.ShapeDtypeStruct((NC, NPAD, D), jnp.float32),
    mesh=_mesh,
    compiler_params=_params,
    scratch_types=[
        pltpu.VMEM((KE,), jnp.int32),
        pltpu.VMEM((KE,), jnp.int32),
        pltpu.VMEM((KE,), jnp.int32),
        pltpu.VMEM((KE,), jnp.int32),
        pltpu.VMEM((KE,), jnp.float32),
        pltpu.VMEM((KE,), jnp.float32),
        pltpu.VMEM((TAIL,), jnp.int32),
        pltpu.VMEM((TAIL,), jnp.int32),
        pltpu.VMEM((TAIL,), jnp.float32),
        pltpu.VMEM((KE, D), jnp.float32),
        pltpu.VMEM((KE, D), jnp.float32),
        pltpu.SemaphoreType.DMA,
        pltpu.SemaphoreType.DMA,
        pltpu.SemaphoreType.DMA,
        pltpu.SemaphoreType.DMA,
        pltpu.SemaphoreType.DMA,
        pltpu.SemaphoreType.DMA,
        pltpu.SemaphoreType.DMA,
        pltpu.VMEM_SHARED((NPAD, D), jnp.float32),
    ],
)
def _sc_propagate(y_hbm, src_hbm, dst_hbm, ew_hbm, out_hbm,
                  srcA, srcB, dstA, dstB, ewA, ewB,
                  srcT, dstT, ewT, rowsA, rowsB,
                  isemA, isemB, gsemA, gsemB, ssemA, ssemB, semT, acc):
    c = lax.axis_index("c")
    s = lax.axis_index("s")
    w = c * NS + s
    e0 = w * EPW

    @pl.loop(0, KE)
    def _(i):
        for r in range(D // L):
            rowsA[i, pl.ds(r * L, L)] = jnp.zeros((L,), jnp.float32)

    @pl.loop(0, RPT // KE)
    def _(k):
        pltpu.sync_copy(rowsA, acc.at[pl.ds(s * RPT + k * KE, KE)])

    plsc.subcore_barrier()

    bufs = ((srcA, dstA, ewA, rowsA, isemA, gsemA, ssemA),
            (srcB, dstB, ewB, rowsB, isemB, gsemB, ssemB))

    def start_chunk(b, t, drain_scatter):
        srcc, dstc, ewc, rows, isem, gsem, ssem = bufs[b]
        if drain_scatter:
            pltpu.make_async_copy(rows, acc.at[dstc], ssem).wait()
        base = e0 + t * KE
        csrc = pltpu.make_async_copy(src_hbm.at[pl.ds(base, KE)], srcc, isem)
        cdst = pltpu.make_async_copy(dst_hbm.at[pl.ds(base, KE)], dstc, isem)
        cew = pltpu.make_async_copy(ew_hbm.at[pl.ds(base, KE)], ewc, isem)
        csrc.start()
        cdst.start()
        cew.start()
        csrc.wait()
        cdst.wait()
        cew.wait()
        pltpu.async_copy(y_hbm.at[srcc], rows, gsem)

    def finish_chunk(b, t):
        srcc, dstc, ewc, rows, isem, gsem, ssem = bufs[b]
        pltpu.make_async_copy(y_hbm.at[srcc], rows, gsem).wait()

        @pl.loop(0, KE)
        def _(j):
            spl = _splat(ewc, j)
            for r in range(D // L):
                rows[j, pl.ds(r * L, L)] = rows[j, pl.ds(r * L, L)] * spl

        pltpu.async_copy(rows, acc.at[dstc], ssem, add=True)

    start_chunk(0, 0, False)
    start_chunk(1, 1, False)

    @pl.loop(0, NFULL, step=2)
    def _(t):
        for b in (0, 1):
            tt = t + b
            finish_chunk(b, tt)

            @pl.when(tt + 2 < NFULL)
            def _():
                start_chunk(b, tt + 2, True)

    pltpu.make_async_copy(rowsA, acc.at[dstA], ssemA).wait()
    pltpu.make_async_copy(rowsB, acc.at[dstB], ssemB).wait()

    tbase = e0 + NFULL * KE
    pltpu.sync_copy(src_hbm.at[pl.ds(tbase, TAIL)], srcT)
    pltpu.sync_copy(dst_hbm.at[pl.ds(tbase, TAIL)], dstT)
    pltpu.sync_copy(ew_hbm.at[pl.ds(tbase, TAIL)], ewT)
    rowsT = rowsA.at[pl.ds(0, TAIL)]
    pltpu.async_copy(y_hbm.at[srcT], rowsT, semT).wait()

    @pl.loop(0, TAIL)
    def _(j):
        spl = _splat(ewT, j)
        for r in range(D // L):
            rowsA[j, pl.ds(r * L, L)] = rowsA[j, pl.ds(r * L, L)] * spl

    pltpu.sync_copy(rowsT, acc.at[dstT], add=True)

    plsc.subcore_barrier()

    @pl.loop(0, RPT // KE)
    def _(k):
        off = s * RPT + k * KE
        pltpu.sync_copy(acc.at[pl.ds(off, KE)], out_hbm.at[c, pl.ds(off, KE)])


def _tc_stage1_body(degp_ref, x_ref, w1_ref, y_ref, dis_ref):
    deg = degp_ref[0] + degp_ref[1] + 1.0
    dis = jnp.where(deg > 0, lax.rsqrt(jnp.maximum(deg, 1e-12)), 0.0)
    y = jnp.dot(x_ref[...], w1_ref[...], preferred_element_type=jnp.float32)
    y_ref[...] = y * dis
    dis_ref[...] = dis


def _tc_stage2_body(agg_ref, y_ref, dis_ref, b1_ref, w2_ref, y2_ref):
    dis = dis_ref[...]
    h = dis * (agg_ref[0, :N] + agg_ref[1, :N] + y_ref[...]) + b1_ref[...]
    h = jnp.maximum(h, 0.0)
    y2 = jnp.dot(h, w2_ref[...], preferred_element_type=jnp.float32)
    y2_ref[...] = y2 * dis


def _tc_stage3_body(agg_ref, y2_ref, dis_ref, bcat_ref, out_ref):
    dis = dis_ref[...]
    out_ref[...] = dis * (agg_ref[0, :N] + agg_ref[1, :N] + y2_ref[...]) \
        + bcat_ref[...]


def _tc_stage1(deg_parts, x, W1):
    return pl.pallas_call(
        _tc_stage1_body,
        out_shape=(
            jax.ShapeDtypeStruct((N, D), jnp.float32),
            jax.ShapeDtypeStruct((N, 1), jnp.float32),
        ),
    )(deg_parts, x, W1)


def _tc_stage2(agg, y, dis, b1, W2):
    return pl.pallas_call(
        _tc_stage2_body,
        out_shape=jax.ShapeDtypeStruct((N, D), jnp.float32),
    )(agg, y, dis, b1, W2)


def _tc_stage3(agg, y2, dis, bcat):
    return pl.pallas_call(
        _tc_stage3_body,
        out_shape=jax.ShapeDtypeStruct((N, D), jnp.float32),
    )(agg, y2, dis, bcat)


@jax.jit
def kernel(x, edge_index, edge_attr, W1, b1, Wmu, bmu, Wlv, blv):
    src = edge_index[0]
    dst = edge_index[1]
    W2 = jnp.concatenate([Wmu, Wlv], axis=1)
    bcat = jnp.concatenate([bmu, blv])[None, :]

    deg_parts = _sc_degree(dst, edge_attr)
    deg_lin = deg_parts.reshape(NC, NPAD)[:, :N, None]
    y1, dis = _tc_stage1(deg_lin, x, W1)
    agg1 = _sc_propagate(y1, src, dst, edge_attr)
    y2 = _tc_stage2(agg1, y1, dis, b1[None, :], W2)
    agg2 = _sc_propagate(y2, src, dst, edge_attr)
    out2 = _tc_stage3(agg2, y2, dis, bcat)
    return out2[:, :D // 2], out2[:, D // 2:]


# speedup vs baseline: 23.2082x; 1.0361x over previous
import functools

import jax
import jax.numpy as jnp
from jax import lax
from jax.experimental import pallas as pl
from jax.experimental.pallas import tpu as pltpu
from jax.experimental.pallas import tpu_sc as plsc

N = 10000
E = 320000
D = 128
L = 16
NC = 2
NS = 16
NW = NC * NS
KE = 128
NCHUNK = E // KE
FULL_T = NCHUNK // NW
REM = NCHUNK - FULL_T * NW
NPAD = 10240
RPT = NPAD // NS
EPW = E // NW
NFULL = EPW // KE
TAIL = EPW - NFULL * KE

_mesh = plsc.VectorSubcoreMesh(core_axis_name="c", subcore_axis_name="s")
_params = pltpu.CompilerParams(needs_layout_passes=False)


def _splat(ewv, j):
    return plsc.load_gather(ewv, [jnp.full((L,), j, jnp.int32)])


NROW = NPAD // D


@functools.partial(
    pl.kernel,
    out_type=jax.ShapeDtypeStruct((NC, NROW, D), jnp.float32),
    mesh=_mesh,
    compiler_params=_params,
    scratch_types=[
        pltpu.VMEM((EPW,), jnp.int32),
        pltpu.VMEM((EPW,), jnp.float32),
        pltpu.VMEM((N * 8,), jnp.float32),
        pltpu.VMEM((NROW, D), jnp.float32),
        pltpu.VMEM((NROW,), jnp.int32),
        pltpu.VMEM_SHARED((NROW, D), jnp.float32),
    ],
)
def _sc_degree(dst_hbm, ew_hbm, out_hbm, dstall, ewall, acc8, stag, idv, acc):
    c = lax.axis_index("c")
    s = lax.axis_index("s")
    w = c * NS + s
    e0 = w * EPW

    pltpu.sync_copy(dst_hbm.at[pl.ds(e0, EPW)], dstall)
    pltpu.sync_copy(ew_hbm.at[pl.ds(e0, EPW)], ewall)

    zero16 = jnp.zeros((L,), jnp.float32)
    lane = lax.broadcasted_iota(jnp.int32, (L,), 0)

    @pl.loop(0, N * 8 // L)
    def _(i):
        acc8[pl.ds(i * L, L)] = zero16

    @pl.loop(0, NROW)
    def _(q):
        for r in range(D // L):
            stag[q, pl.ds(r * L, L)] = zero16

    @pl.loop(0, NROW // L)
    def _(g):
        idv[pl.ds(g * L, L)] = lane + g * L

    @pl.when(s == 0)
    def _():
        pltpu.sync_copy(stag, acc)

    plsc.subcore_barrier()

    m_lo = lane < 8
    m_hi = lane >= 8
    col_lo = lane
    col_hi = lane - 8

    @pl.loop(0, EPW // L)
    def _(g):
        gofs = pl.ds(g * L, L)
        dstv = dstall[gofs]
        ewv = ewall[gofs]
        base8 = dstv * 8
        plsc.addupdate_scatter(acc8, [base8 + col_lo], ewv, mask=m_lo)
        plsc.addupdate_scatter(acc8, [base8 + col_hi], ewv, mask=m_hi)

    @pl.loop(0, N // L)
    def _(m):
        tot = zero16
        for cc in range(8):
            tot = tot + plsc.load_gather(acc8, [(lane + m * L) * 8 + cc])
        stag[m // 8, pl.ds((m % 8) * L, L)] = tot

    pltpu.sync_copy(stag, acc.at[idv], add=True)
    plsc.subcore_barrier()

    @pl.when(s < NROW // 8)
    def _():
        pltpu.sync_copy(acc.at[pl.ds(s * 8, 8)], out_hbm.at[c, pl.ds(s * 8, 8)])


@functools.partial(
    pl.kernel,
    out_type=jax.ShapeDtypeStruct((NC, NPAD, D), jnp.float32),
    mesh=_mesh,
    compiler_params=_params,
    scratch_types=[
        pltpu.VMEM((KE,), jnp.int32),
        pltpu.VMEM((KE,), jnp.int32),
        pltpu.VMEM((KE,), jnp.int32),
        pltpu.VMEM((KE,), jnp.int32),
        pltpu.VMEM((KE,), jnp.float32),
        pltpu.VMEM((KE,), jnp.float32),
        pltpu.VMEM((TAIL,), jnp.int32),
        pltpu.VMEM((TAIL,), jnp.int32),
        pltpu.VMEM((TAIL,), jnp.float32),
        pltpu.VMEM((KE, D), jnp.float32),
        pltpu.VMEM((KE, D), jnp.float32),
        pltpu.SemaphoreType.DMA,
        pltpu.SemaphoreType.DMA,
        pltpu.SemaphoreType.DMA,
        pltpu.SemaphoreType.DMA,
        pltpu.SemaphoreType.DMA,
        pltpu.SemaphoreType.DMA,
        pltpu.SemaphoreType.DMA,
        pltpu.VMEM_SHARED((NPAD, D), jnp.float32),
    ],
)
def _sc_propagate(y_hbm, src_hbm, dst_hbm, ew_hbm, out_hbm,
                  srcA, srcB, dstA, dstB, ewA, ewB,
                  srcT, dstT, ewT, rowsA, rowsB,
                  isemA, isemB, gsemA, gsemB, ssemA, ssemB, semT, acc):
    c = lax.axis_index("c")
    s = lax.axis_index("s")
    w = c * NS + s
    e0 = w * EPW

    @pl.loop(0, KE)
    def _(i):
        for r in range(D // L):
            rowsA[i, pl.ds(r * L, L)] = jnp.zeros((L,), jnp.float32)

    @pl.loop(0, RPT // KE)
    def _(k):
        pltpu.sync_copy(rowsA, acc.at[pl.ds(s * RPT + k * KE, KE)])

    plsc.subcore_barrier()

    bufs = ((srcA, dstA, ewA, rowsA, isemA, gsemA, ssemA),
            (srcB, dstB, ewB, rowsB, isemB, gsemB, ssemB))

    def start_chunk(b, t, drain_scatter):
        srcc, dstc, ewc, rows, isem, gsem, ssem = bufs[b]
        if drain_scatter:
            pltpu.make_async_copy(rows, acc.at[dstc], ssem).wait()
        base = e0 + t * KE
        csrc = pltpu.make_async_copy(src_hbm.at[pl.ds(base, KE)], srcc, isem)
        cdst = pltpu.make_async_copy(dst_hbm.at[pl.ds(base, KE)], dstc, isem)
        cew = pltpu.make_async_copy(ew_hbm.at[pl.ds(base, KE)], ewc, isem)
        csrc.start()
        cdst.start()
        cew.start()
        csrc.wait()
        cdst.wait()
        cew.wait()
        pltpu.async_copy(y_hbm.at[srcc], rows, gsem)

    def finish_chunk(b, t):
        srcc, dstc, ewc, rows, isem, gsem, ssem = bufs[b]
        pltpu.make_async_copy(y_hbm.at[srcc], rows, gsem).wait()

        @pl.loop(0, KE, unroll=4)
        def _(j):
            spl = _splat(ewc, j)
            for r in range(D // L):
                rows[j, pl.ds(r * L, L)] = rows[j, pl.ds(r * L, L)] * spl

        pltpu.async_copy(rows, acc.at[dstc], ssem, add=True)

    start_chunk(0, 0, False)
    start_chunk(1, 1, False)

    @pl.loop(0, NFULL, step=2)
    def _(t):
        for b in (0, 1):
            tt = t + b
            finish_chunk(b, tt)

            @pl.when(tt + 2 < NFULL)
            def _():
                start_chunk(b, tt + 2, True)

    pltpu.make_async_copy(rowsA, acc.at[dstA], ssemA).wait()
    pltpu.make_async_copy(rowsB, acc.at[dstB], ssemB).wait()

    tbase = e0 + NFULL * KE
    pltpu.sync_copy(src_hbm.at[pl.ds(tbase, TAIL)], srcT)
    pltpu.sync_copy(dst_hbm.at[pl.ds(tbase, TAIL)], dstT)
    pltpu.sync_copy(ew_hbm.at[pl.ds(tbase, TAIL)], ewT)
    rowsT = rowsA.at[pl.ds(0, TAIL)]
    pltpu.async_copy(y_hbm.at[srcT], rowsT, semT).wait()

    @pl.loop(0, TAIL)
    def _(j):
        spl = _splat(ewT, j)
        for r in range(D // L):
            rowsA[j, pl.ds(r * L, L)] = rowsA[j, pl.ds(r * L, L)] * spl

    pltpu.sync_copy(rowsT, acc.at[dstT], add=True)

    plsc.subcore_barrier()

    @pl.loop(0, RPT // KE)
    def _(k):
        off = s * RPT + k * KE
        pltpu.sync_copy(acc.at[pl.ds(off, KE)], out_hbm.at[c, pl.ds(off, KE)])


def _tc_stage1_body(degp_ref, x_ref, w1_ref, y_ref, dis_ref):
    deg = degp_ref[0] + degp_ref[1] + 1.0
    dis = jnp.where(deg > 0, lax.rsqrt(jnp.maximum(deg, 1e-12)), 0.0)
    y = jnp.dot(x_ref[...], w1_ref[...], preferred_element_type=jnp.float32)
    y_ref[...] = y * dis
    dis_ref[...] = dis


def _tc_stage2_body(agg_ref, y_ref, dis_ref, b1_ref, w2_ref, y2_ref):
    dis = dis_ref[...]
    h = dis * (agg_ref[0, :N] + agg_ref[1, :N] + y_ref[...]) + b1_ref[...]
    h = jnp.maximum(h, 0.0)
    y2 = jnp.dot(h, w2_ref[...], preferred_element_type=jnp.float32)
    y2_ref[...] = y2 * dis


def _tc_stage3_body(agg_ref, y2_ref, dis_ref, bcat_ref, out_ref):
    dis = dis_ref[...]
    out_ref[...] = dis * (agg_ref[0, :N] + agg_ref[1, :N] + y2_ref[...]) \
        + bcat_ref[...]


def _tc_stage1(deg_parts, x, W1):
    return pl.pallas_call(
        _tc_stage1_body,
        out_shape=(
            jax.ShapeDtypeStruct((N, D), jnp.float32),
            jax.ShapeDtypeStruct((N, 1), jnp.float32),
        ),
    )(deg_parts, x, W1)


def _tc_stage2(agg, y, dis, b1, W2):
    return pl.pallas_call(
        _tc_stage2_body,
        out_shape=jax.ShapeDtypeStruct((N, D), jnp.float32),
    )(agg, y, dis, b1, W2)


def _tc_stage3(agg, y2, dis, bcat):
    return pl.pallas_call(
        _tc_stage3_body,
        out_shape=jax.ShapeDtypeStruct((N, D), jnp.float32),
    )(agg, y2, dis, bcat)


@jax.jit
def kernel(x, edge_index, edge_attr, W1, b1, Wmu, bmu, Wlv, blv):
    src = edge_index[0]
    dst = edge_index[1]
    W2 = jnp.concatenate([Wmu, Wlv], axis=1)
    bcat = jnp.concatenate([bmu, blv])[None, :]

    deg_parts = _sc_degree(dst, edge_attr)
    deg_lin = deg_parts.reshape(NC, NPAD)[:, :N, None]
    y1, dis = _tc_stage1(deg_lin, x, W1)
    agg1 = _sc_propagate(y1, src, dst, edge_attr)
    y2 = _tc_stage2(agg1, y1, dis, b1[None, :], W2)
    agg2 = _sc_propagate(y2, src, dst, edge_attr)
    out2 = _tc_stage3(agg2, y2, dis, bcat)
    return out2[:, :D // 2], out2[:, D // 2:]


# speedup vs baseline: 23.3838x; 1.0076x over previous
import functools

import jax
import jax.numpy as jnp
from jax import lax
from jax.experimental import pallas as pl
from jax.experimental.pallas import tpu as pltpu
from jax.experimental.pallas import tpu_sc as plsc

N = 10000
E = 320000
D = 128
L = 16
NC = 2
NS = 16
NW = NC * NS
KE = 128
NCHUNK = E // KE
FULL_T = NCHUNK // NW
REM = NCHUNK - FULL_T * NW
NPAD = 10240
RPT = NPAD // NS
EPW = E // NW
NFULL = EPW // KE
TAIL = EPW - NFULL * KE

_mesh = plsc.VectorSubcoreMesh(core_axis_name="c", subcore_axis_name="s")
_params = pltpu.CompilerParams(needs_layout_passes=False)


def _splat(ewv, j):
    return plsc.load_gather(ewv, [jnp.full((L,), j, jnp.int32)])


NROW = NPAD // D


@functools.partial(
    pl.kernel,
    out_type=jax.ShapeDtypeStruct((NC, NROW, D), jnp.float32),
    mesh=_mesh,
    compiler_params=_params,
    scratch_types=[
        pltpu.VMEM((EPW,), jnp.int32),
        pltpu.VMEM((EPW,), jnp.float32),
        pltpu.VMEM((N * 8,), jnp.float32),
        pltpu.VMEM((NROW, D), jnp.float32),
        pltpu.VMEM((NROW,), jnp.int32),
        pltpu.VMEM_SHARED((NROW, D), jnp.float32),
    ],
)
def _sc_degree(dst_hbm, ew_hbm, out_hbm, dstall, ewall, acc8, stag, idv, acc):
    c = lax.axis_index("c")
    s = lax.axis_index("s")
    w = c * NS + s
    e0 = w * EPW

    pltpu.sync_copy(dst_hbm.at[pl.ds(e0, EPW)], dstall)
    pltpu.sync_copy(ew_hbm.at[pl.ds(e0, EPW)], ewall)

    zero16 = jnp.zeros((L,), jnp.float32)
    lane = lax.broadcasted_iota(jnp.int32, (L,), 0)

    @pl.loop(0, N * 8 // L)
    def _(i):
        acc8[pl.ds(i * L, L)] = zero16

    @pl.loop(0, NROW)
    def _(q):
        for r in range(D // L):
            stag[q, pl.ds(r * L, L)] = zero16

    @pl.loop(0, NROW // L)
    def _(g):
        idv[pl.ds(g * L, L)] = lane + g * L

    @pl.when(s == 0)
    def _():
        pltpu.sync_copy(stag, acc)

    plsc.subcore_barrier()

    m_lo = lane < 8
    m_hi = lane >= 8
    col_lo = lane
    col_hi = lane - 8

    @pl.loop(0, EPW // L, unroll=4)
    def _(g):
        gofs = pl.ds(g * L, L)
        dstv = dstall[gofs]
        ewv = ewall[gofs]
        base8 = dstv * 8
        plsc.addupdate_scatter(acc8, [base8 + col_lo], ewv, mask=m_lo)
        plsc.addupdate_scatter(acc8, [base8 + col_hi], ewv, mask=m_hi)

    @pl.loop(0, N // L, unroll=2)
    def _(m):
        tot = zero16
        for cc in range(8):
            tot = tot + plsc.load_gather(acc8, [(lane + m * L) * 8 + cc])
        stag[m // 8, pl.ds((m % 8) * L, L)] = tot

    pltpu.sync_copy(stag, acc.at[idv], add=True)
    plsc.subcore_barrier()

    @pl.when(s < NROW // 8)
    def _():
        pltpu.sync_copy(acc.at[pl.ds(s * 8, 8)], out_hbm.at[c, pl.ds(s * 8, 8)])


@functools.partial(
    pl.kernel,
    out_type=jax.ShapeDtypeStruct((NC, NPAD, D), jnp.float32),
    mesh=_mesh,
    compiler_params=_params,
    scratch_types=[
        pltpu.VMEM((KE,), jnp.int32),
        pltpu.VMEM((KE,), jnp.int32),
        pltpu.VMEM((KE,), jnp.int32),
        pltpu.VMEM((KE,), jnp.int32),
        pltpu.VMEM((KE,), jnp.float32),
        pltpu.VMEM((KE,), jnp.float32),
        pltpu.VMEM((TAIL,), jnp.int32),
        pltpu.VMEM((TAIL,), jnp.int32),
        pltpu.VMEM((TAIL,), jnp.float32),
        pltpu.VMEM((KE, D), jnp.float32),
        pltpu.VMEM((KE, D), jnp.float32),
        pltpu.SemaphoreType.DMA,
        pltpu.SemaphoreType.DMA,
        pltpu.SemaphoreType.DMA,
        pltpu.SemaphoreType.DMA,
        pltpu.SemaphoreType.DMA,
        pltpu.SemaphoreType.DMA,
        pltpu.SemaphoreType.DMA,
        pltpu.VMEM_SHARED((NPAD, D), jnp.float32),
    ],
)
def _sc_propagate(y_hbm, src_hbm, dst_hbm, ew_hbm, out_hbm,
                  srcA, srcB, dstA, dstB, ewA, ewB,
                  srcT, dstT, ewT, rowsA, rowsB,
                  isemA, isemB, gsemA, gsemB, ssemA, ssemB, semT, acc):
    c = lax.axis_index("c")
    s = lax.axis_index("s")
    w = c * NS + s
    e0 = w * EPW

    @pl.loop(0, KE)
    def _(i):
        for r in range(D // L):
            rowsA[i, pl.ds(r * L, L)] = jnp.zeros((L,), jnp.float32)

    @pl.loop(0, RPT // KE)
    def _(k):
        pltpu.sync_copy(rowsA, acc.at[pl.ds(s * RPT + k * KE, KE)])

    plsc.subcore_barrier()

    bufs = ((srcA, dstA, ewA, rowsA, isemA, gsemA, ssemA),
            (srcB, dstB, ewB, rowsB, isemB, gsemB, ssemB))

    def start_chunk(b, t, drain_scatter):
        srcc, dstc, ewc, rows, isem, gsem, ssem = bufs[b]
        if drain_scatter:
            pltpu.make_async_copy(rows, acc.at[dstc], ssem).wait()
        base = e0 + t * KE
        csrc = pltpu.make_async_copy(src_hbm.at[pl.ds(base, KE)], srcc, isem)
        cdst = pltpu.make_async_copy(dst_hbm.at[pl.ds(base, KE)], dstc, isem)
        cew = pltpu.make_async_copy(ew_hbm.at[pl.ds(base, KE)], ewc, isem)
        csrc.start()
        cdst.start()
        cew.start()
        csrc.wait()
        cdst.wait()
        cew.wait()
        pltpu.async_copy(y_hbm.at[srcc], rows, gsem)

    def finish_chunk(b, t):
        srcc, dstc, ewc, rows, isem, gsem, ssem = bufs[b]
        pltpu.make_async_copy(y_hbm.at[srcc], rows, gsem).wait()

        @pl.loop(0, KE, unroll=4)
        def _(j):
            spl = _splat(ewc, j)
            for r in range(D // L):
                rows[j, pl.ds(r * L, L)] = rows[j, pl.ds(r * L, L)] * spl

        pltpu.async_copy(rows, acc.at[dstc], ssem, add=True)

    start_chunk(0, 0, False)
    start_chunk(1, 1, False)

    @pl.loop(0, NFULL, step=2)
    def _(t):
        for b in (0, 1):
            tt = t + b
            finish_chunk(b, tt)

            @pl.when(tt + 2 < NFULL)
            def _():
                start_chunk(b, tt + 2, True)

    pltpu.make_async_copy(rowsA, acc.at[dstA], ssemA).wait()
    pltpu.make_async_copy(rowsB, acc.at[dstB], ssemB).wait()

    tbase = e0 + NFULL * KE
    pltpu.sync_copy(src_hbm.at[pl.ds(tbase, TAIL)], srcT)
    pltpu.sync_copy(dst_hbm.at[pl.ds(tbase, TAIL)], dstT)
    pltpu.sync_copy(ew_hbm.at[pl.ds(tbase, TAIL)], ewT)
    rowsT = rowsA.at[pl.ds(0, TAIL)]
    pltpu.async_copy(y_hbm.at[srcT], rowsT, semT).wait()

    @pl.loop(0, TAIL)
    def _(j):
        spl = _splat(ewT, j)
        for r in range(D // L):
            rowsA[j, pl.ds(r * L, L)] = rowsA[j, pl.ds(r * L, L)] * spl

    pltpu.sync_copy(rowsT, acc.at[dstT], add=True)

    plsc.subcore_barrier()

    @pl.loop(0, RPT // KE)
    def _(k):
        off = s * RPT + k * KE
        pltpu.sync_copy(acc.at[pl.ds(off, KE)], out_hbm.at[c, pl.ds(off, KE)])


def _tc_stage1_body(degp_ref, x_ref, w1_ref, y_ref, dis_ref):
    deg = degp_ref[0] + degp_ref[1] + 1.0
    dis = jnp.where(deg > 0, lax.rsqrt(jnp.maximum(deg, 1e-12)), 0.0)
    y = jnp.dot(x_ref[...], w1_ref[...], preferred_element_type=jnp.float32)
    y_ref[...] = y * dis
    dis_ref[...] = dis


def _tc_stage2_body(agg_ref, y_ref, dis_ref, b1_ref, w2_ref, y2_ref):
    dis = dis_ref[...]
    h = dis * (agg_ref[0, :N] + agg_ref[1, :N] + y_ref[...]) + b1_ref[...]
    h = jnp.maximum(h, 0.0)
    y2 = jnp.dot(h, w2_ref[...], preferred_element_type=jnp.float32)
    y2_ref[...] = y2 * dis


def _tc_stage3_body(agg_ref, y2_ref, dis_ref, bcat_ref, mu_ref, lv_ref):
    dis = dis_ref[...]
    out2 = dis * (agg_ref[0, :N] + agg_ref[1, :N] + y2_ref[...]) \
        + bcat_ref[...]
    mu_ref[...] = out2[:, :D // 2]
    lv_ref[...] = out2[:, D // 2:]


def _tc_stage1(deg_parts, x, W1):
    return pl.pallas_call(
        _tc_stage1_body,
        out_shape=(
            jax.ShapeDtypeStruct((N, D), jnp.float32),
            jax.ShapeDtypeStruct((N, 1), jnp.float32),
        ),
    )(deg_parts, x, W1)


def _tc_stage2(agg, y, dis, b1, W2):
    return pl.pallas_call(
        _tc_stage2_body,
        out_shape=jax.ShapeDtypeStruct((N, D), jnp.float32),
    )(agg, y, dis, b1, W2)


def _tc_stage3(agg, y2, dis, bcat):
    return pl.pallas_call(
        _tc_stage3_body,
        out_shape=(
            jax.ShapeDtypeStruct((N, D // 2), jnp.float32),
            jax.ShapeDtypeStruct((N, D // 2), jnp.float32),
        ),
    )(agg, y2, dis, bcat)


@jax.jit
def kernel(x, edge_index, edge_attr, W1, b1, Wmu, bmu, Wlv, blv):
    src = edge_index[0]
    dst = edge_index[1]
    W2 = jnp.concatenate([Wmu, Wlv], axis=1)
    bcat = jnp.concatenate([bmu, blv])[None, :]

    deg_parts = _sc_degree(dst, edge_attr)
    deg_lin = deg_parts.reshape(NC, NPAD)[:, :N, None]
    y1, dis = _tc_stage1(deg_lin, x, W1)
    agg1 = _sc_propagate(y1, src, dst, edge_attr)
    y2 = _tc_stage2(agg1, y1, dis, b1[None, :], W2)
    agg2 = _sc_propagate(y2, src, dst, edge_attr)
    mu, logvar = _tc_stage3(agg2, y2, dis, bcat)
    return mu, logvar
